# baseline (speedup 1.0000x reference)
import jax
import jax.numpy as jnp
from jax.experimental import pallas as pl

_N = 10001
_D = 256
_M_BLK = 256


def _gcn_kernel(a_ref, x_ref, o_ref, obf_ref):
    a = a_ref[...].astype(jnp.bfloat16)
    acc = jnp.dot(a, x_ref[...], preferred_element_type=jnp.float32)
    o_ref[...] = acc
    obf_ref[...] = acc.astype(jnp.bfloat16)


def _gcn_last_kernel(a_ref, x_ref, e0_ref, e1_ref, e2_ref, o_ref, tot_ref):
    a = a_ref[...].astype(jnp.bfloat16)
    acc = jnp.dot(a, x_ref[...], preferred_element_type=jnp.float32)
    o_ref[...] = acc
    tot_ref[...] = e0_ref[...] + e1_ref[...] + e2_ref[...] + acc


def _emb_spec():
    return pl.BlockSpec((_M_BLK, _D), lambda i: (i, 0))


def kernel(encoder_adj, item_emb):
    nm = pl.cdiv(_N, _M_BLK)
    a_spec = pl.BlockSpec((_M_BLK, _N), lambda i: (i, 0))
    x_spec = pl.BlockSpec((_N, _D), lambda i: (0, 0))

    mm = pl.pallas_call(
        _gcn_kernel,
        grid=(nm,),
        in_specs=[a_spec, x_spec],
        out_specs=[_emb_spec(), _emb_spec()],
        out_shape=[
            jax.ShapeDtypeStruct((_N, _D), jnp.float32),
            jax.ShapeDtypeStruct((_N, _D), jnp.bfloat16),
        ],
    )
    mm_last = pl.pallas_call(
        _gcn_last_kernel,
        grid=(nm,),
        in_specs=[a_spec, x_spec, _emb_spec(), _emb_spec(), _emb_spec()],
        out_specs=[_emb_spec(), _emb_spec()],
        out_shape=[
            jax.ShapeDtypeStruct((_N, _D), jnp.float32),
            jax.ShapeDtypeStruct((_N, _D), jnp.float32),
        ],
    )

    x0_bf = item_emb.astype(jnp.bfloat16)
    e1, e1_bf = mm(encoder_adj, x0_bf)
    e2, e2_bf = mm(encoder_adj, e1_bf)
    e3, total = mm_last(encoder_adj, e2_bf, item_emb, e1, e2)
    return (total, (item_emb, e1, e2, e3))


# speedup vs baseline: 1.1076x; 1.1076x over previous
import jax
import jax.numpy as jnp
from jax.experimental import pallas as pl

_N = 10001
_D = 256
_M_BLK1 = 256
_M_BLK = 512


def _gcn_first_kernel(a_ref, x_ref, o_ref, obf_ref, abf_ref):
    a = a_ref[...].astype(jnp.bfloat16)
    abf_ref[...] = a
    acc = jnp.dot(a, x_ref[...], preferred_element_type=jnp.float32)
    o_ref[...] = acc
    obf_ref[...] = acc.astype(jnp.bfloat16)


def _gcn_kernel(a_ref, x_ref, o_ref, obf_ref):
    acc = jnp.dot(a_ref[...], x_ref[...], preferred_element_type=jnp.float32)
    o_ref[...] = acc
    obf_ref[...] = acc.astype(jnp.bfloat16)


def _gcn_last_kernel(a_ref, x_ref, e0_ref, e1_ref, e2_ref, o_ref, tot_ref):
    acc = jnp.dot(a_ref[...], x_ref[...], preferred_element_type=jnp.float32)
    o_ref[...] = acc
    tot_ref[...] = e0_ref[...] + e1_ref[...] + e2_ref[...] + acc


def kernel(encoder_adj, item_emb):
    x_spec = pl.BlockSpec((_N, _D), lambda i: (0, 0))

    nm1 = pl.cdiv(_N, _M_BLK1)
    a1_spec = pl.BlockSpec((_M_BLK1, _N), lambda i: (i, 0))
    e1_spec = pl.BlockSpec((_M_BLK1, _D), lambda i: (i, 0))
    mm_first = pl.pallas_call(
        _gcn_first_kernel,
        grid=(nm1,),
        in_specs=[a1_spec, x_spec],
        out_specs=[e1_spec, e1_spec, a1_spec],
        out_shape=[
            jax.ShapeDtypeStruct((_N, _D), jnp.float32),
            jax.ShapeDtypeStruct((_N, _D), jnp.bfloat16),
            jax.ShapeDtypeStruct((_N, _N), jnp.bfloat16),
        ],
    )

    nm = pl.cdiv(_N, _M_BLK)
    a_spec = pl.BlockSpec((_M_BLK, _N), lambda i: (i, 0))
    e_spec = pl.BlockSpec((_M_BLK, _D), lambda i: (i, 0))
    mm = pl.pallas_call(
        _gcn_kernel,
        grid=(nm,),
        in_specs=[a_spec, x_spec],
        out_specs=[e_spec, e_spec],
        out_shape=[
            jax.ShapeDtypeStruct((_N, _D), jnp.float32),
            jax.ShapeDtypeStruct((_N, _D), jnp.bfloat16),
        ],
    )
    mm_last = pl.pallas_call(
        _gcn_last_kernel,
        grid=(nm,),
        in_specs=[a_spec, x_spec, e_spec, e_spec, e_spec],
        out_specs=[e_spec, e_spec],
        out_shape=[
            jax.ShapeDtypeStruct((_N, _D), jnp.float32),
            jax.ShapeDtypeStruct((_N, _D), jnp.float32),
        ],
    )

    x0_bf = item_emb.astype(jnp.bfloat16)
    e1, e1_bf, a_bf = mm_first(encoder_adj, x0_bf)
    e2, e2_bf = mm(a_bf, e1_bf)
    e3, total = mm_last(a_bf, e2_bf, item_emb, e1, e2)
    return (total, (item_emb, e1, e2, e3))


# speedup vs baseline: 1.1389x; 1.0283x over previous
import jax
import jax.numpy as jnp
from jax.experimental import pallas as pl
from jax.experimental.pallas import tpu as pltpu

_N = 10001
_D = 256
_M_BLK1 = 256
_M_BLK = 1024


def _gcn_first_kernel(a_ref, x_ref, o_ref, obf_ref, abf_ref):
    a = a_ref[...].astype(jnp.bfloat16)
    abf_ref[...] = a
    x = x_ref[...].astype(jnp.bfloat16)
    acc = jnp.dot(a, x, preferred_element_type=jnp.float32)
    o_ref[...] = acc
    obf_ref[...] = acc.astype(jnp.bfloat16)


def _gcn_kernel(a_ref, x_ref, o_ref, obf_ref):
    acc = jnp.dot(a_ref[...], x_ref[...], preferred_element_type=jnp.float32)
    o_ref[...] = acc
    obf_ref[...] = acc.astype(jnp.bfloat16)


def _gcn_last_kernel(a_ref, x_ref, e0_ref, e1_ref, e2_ref, o_ref, tot_ref):
    acc = jnp.dot(a_ref[...], x_ref[...], preferred_element_type=jnp.float32)
    o_ref[...] = acc
    tot_ref[...] = e0_ref[...] + e1_ref[...] + e2_ref[...] + acc


def kernel(encoder_adj, item_emb):
    x_spec = pl.BlockSpec((_N, _D), lambda i: (0, 0))
    params = pltpu.CompilerParams(dimension_semantics=("parallel",))

    nm1 = pl.cdiv(_N, _M_BLK1)
    a1_spec = pl.BlockSpec((_M_BLK1, _N), lambda i: (i, 0))
    e1_spec = pl.BlockSpec((_M_BLK1, _D), lambda i: (i, 0))
    mm_first = pl.pallas_call(
        _gcn_first_kernel,
        grid=(nm1,),
        in_specs=[a1_spec, x_spec],
        out_specs=[e1_spec, e1_spec, a1_spec],
        out_shape=[
            jax.ShapeDtypeStruct((_N, _D), jnp.float32),
            jax.ShapeDtypeStruct((_N, _D), jnp.bfloat16),
            jax.ShapeDtypeStruct((_N, _N), jnp.bfloat16),
        ],
        compiler_params=params,
    )

    nm = pl.cdiv(_N, _M_BLK)
    a_spec = pl.BlockSpec((_M_BLK, _N), lambda i: (i, 0))
    e_spec = pl.BlockSpec((_M_BLK, _D), lambda i: (i, 0))
    mm = pl.pallas_call(
        _gcn_kernel,
        grid=(nm,),
        in_specs=[a_spec, x_spec],
        out_specs=[e_spec, e_spec],
        out_shape=[
            jax.ShapeDtypeStruct((_N, _D), jnp.float32),
            jax.ShapeDtypeStruct((_N, _D), jnp.bfloat16),
        ],
        compiler_params=params,
    )
    mm_last = pl.pallas_call(
        _gcn_last_kernel,
        grid=(nm,),
        in_specs=[a_spec, x_spec, e_spec, e_spec, e_spec],
        out_specs=[e_spec, e_spec],
        out_shape=[
            jax.ShapeDtypeStruct((_N, _D), jnp.float32),
            jax.ShapeDtypeStruct((_N, _D), jnp.float32),
        ],
        compiler_params=params,
    )

    e1, e1_bf, a_bf = mm_first(encoder_adj, item_emb)
    e2, e2_bf = mm(a_bf, e1_bf)
    e3, total = mm_last(a_bf, e2_bf, item_emb, e1, e2)
    return (total, (item_emb, e1, e2, e3))
